# baseline (speedup 1.0000x reference)
import numpy as np
import jax
import jax.numpy as jnp
from jax import lax
from jax.experimental import pallas as pl
from jax.experimental.pallas import tpu as pltpu
from jax.experimental.pallas import tpu_sc as plsc

VOCAB = 1000000
D = 64
SEQ = 200
BATCH = 4096
NC = 2
NS = 16
NW = NC * NS
ROWS_PER_W = BATCH // NW
SPLIT_A = 120
SPLIT_B = 80
NBUF_IN = 4
NBUF_OUT = 2
SCALE = 8.0


def _pos_encoding(length, depth):
    half = depth // 2
    positions = np.arange(length)[:, np.newaxis]
    depths = np.arange(half)[np.newaxis, :] / half
    angle_rates = 1 / 10000 ** depths
    angle_rads = positions * angle_rates
    pe = np.concatenate([np.sin(angle_rads), np.cos(angle_rads)], axis=-1)
    return pe.astype(np.float32)


def _sc_kernel(table, xa, xb, pos_h, out, idxa, idxb, bin_, bout, pos_v,
               gsem, ssem):
    cid = lax.axis_index("c")
    sid = lax.axis_index("s")
    wid = sid * NC + cid
    r0 = wid * ROWS_PER_W

    pltpu.sync_copy(pos_h, pos_v)
    pltpu.sync_copy(xa.at[pl.ds(r0, ROWS_PER_W)], idxa)
    pltpu.sync_copy(xb.at[pl.ds(r0, ROWS_PER_W)], idxb)

    def gather_a(g, b):
        return pltpu.make_async_copy(
            table.at[idxa.at[g]], bin_.at[b, pl.ds(0, SPLIT_A)], gsem.at[b])

    def gather_b(g, b):
        return pltpu.make_async_copy(
            table.at[idxb.at[g]], bin_.at[b, pl.ds(SPLIT_A, SPLIT_B)],
            gsem.at[b])

    def scatter(g, ob):
        return pltpu.make_async_copy(
            bout.at[ob], out.at[pl.ds((r0 + g) * SEQ, SEQ)], ssem.at[ob])

    for b in range(NBUF_IN):
        gather_a(b, b).start()
        gather_b(b, b).start()

    def outer(k, carry):
        g0 = k * NBUF_IN
        for b in range(NBUF_IN):
            g = g0 + b
            ob = b % NBUF_OUT
            gather_a(g, b).wait()
            gather_b(g, b).wait()

            @pl.when(g >= NBUF_OUT)
            def _():
                scatter(g - NBUF_OUT, ob).wait()

            def row(i, c):
                for d in range(D // 16):
                    s = pl.ds(d * 16, 16)
                    bout[ob, i, s] = bin_[b, i, s] * SCALE + pos_v[i, s]
                return c

            lax.fori_loop(0, SEQ, row, 0)
            scatter(g, ob).start()

            @pl.when(g + NBUF_IN < ROWS_PER_W)
            def _():
                gather_a(g + NBUF_IN, b).start()
                gather_b(g + NBUF_IN, b).start()
        return carry

    lax.fori_loop(0, ROWS_PER_W // NBUF_IN, outer, 0)

    for t in range(NBUF_OUT):
        g = ROWS_PER_W - NBUF_OUT + t
        scatter(g, g % NBUF_OUT).wait()


def kernel(x, embedding_table):
    x = x.astype(jnp.int32)
    xa = x[:, :SPLIT_A]
    xb = x[:, SPLIT_A:]
    pos = jnp.asarray(_pos_encoding(2048, D)[:SEQ])

    mesh = plsc.VectorSubcoreMesh(
        core_axis_name="c", subcore_axis_name="s",
        num_cores=NC, num_subcores=NS)

    out = pl.kernel(
        _sc_kernel,
        out_type=jax.ShapeDtypeStruct((BATCH * SEQ, D), jnp.float32),
        mesh=mesh,
        compiler_params=pltpu.CompilerParams(use_tc_tiling_on_sc=False),
        scratch_types=[
            pltpu.VMEM((ROWS_PER_W, SPLIT_A), jnp.int32),
            pltpu.VMEM((ROWS_PER_W, SPLIT_B), jnp.int32),
            pltpu.VMEM((NBUF_IN, SEQ, D), jnp.float32),
            pltpu.VMEM((NBUF_OUT, SEQ, D), jnp.float32),
            pltpu.VMEM((SEQ, D), jnp.float32),
            pltpu.SemaphoreType.DMA((NBUF_IN,)),
            pltpu.SemaphoreType.DMA((NBUF_OUT,)),
        ],
    )(embedding_table, xa, xb, pos)

    return out.reshape(BATCH, SEQ, D)
